# baseline (speedup 1.0000x reference)
import jax
import jax.numpy as jnp
from jax import lax
from jax.experimental import pallas as pl
from jax.experimental.pallas import tpu as pltpu
from jax.experimental.pallas import tpu_sc as plsc

VOCAB = 100000
D = 128
B = 16384
K = 20
NC = 2
NS = 16
NW = NC * NS
EPW = B // NW
C = 16
NCHUNK = EPW // C
NSUB = 4
SUBN = C * K // NSUB
L = 16


def _body(center_hbm, context_hbm, neg_hbm, win_hbm, wout_hbm,
          pos_hbm, negout_hbm,
          cidx, xidx, nidx2d0, nidx2d1, nflat0, nflat1,
          crows0, crows1, xrows0, xrows1,
          nrows0, nrows1, pos_v0, pos_v1, neg_v0, neg_v1,
          gsem0, gsem1, osem0, osem1):
    wid = lax.axis_index("s") * NC + lax.axis_index("c")
    base0 = wid * EPW
    nidx2d = (nidx2d0, nidx2d1)
    nflat = (nflat0, nflat1)
    crows = (crows0, crows1)
    xrows = (xrows0, xrows1)
    nrows = (nrows0, nrows1)
    pos_v = (pos_v0, pos_v1)
    neg_v = (neg_v0, neg_v1)
    gsems = (gsem0, gsem1)
    osems = (osem0, osem1)

    pltpu.sync_copy(center_hbm.at[pl.ds(base0, EPW)], cidx)
    pltpu.sync_copy(context_hbm.at[pl.ds(base0, EPW)], xidx)

    lane0 = lax.iota(jnp.int32, L)

    def fire(c, b):
        off = c * C
        base = base0 + off
        pltpu.sync_copy(neg_hbm.at[pl.ds(base, C), :], nidx2d[b])
        for e in range(C):
            row = jnp.full((L,), e, jnp.int32)
            g1 = plsc.load_gather(nidx2d[b], [row, lane0])
            plsc.store_scatter(nflat[b], [e * K + lane0], g1)
            tail = 16 + (lane0 & 3)
            g2 = plsc.load_gather(nidx2d[b], [row, tail])
            plsc.store_scatter(nflat[b], [e * K + tail], g2,
                               mask=lane0 < K - 16)
        pltpu.async_copy(win_hbm.at[cidx.at[pl.ds(off, C)]],
                         crows[b], gsems[b])
        pltpu.async_copy(wout_hbm.at[xidx.at[pl.ds(off, C)]],
                         xrows[b], gsems[b])
        for j in range(NSUB):
            pltpu.async_copy(
                wout_hbm.at[nflat[b].at[pl.ds(j * SUBN, SUBN)]],
                nrows[b].at[pl.ds(j * SUBN, SUBN)], gsems[b])

    def wait_gathers(b):
        pltpu.make_async_copy(win_hbm.at[pl.ds(0, C)], crows[b],
                              gsems[b]).wait()
        pltpu.make_async_copy(wout_hbm.at[pl.ds(0, C)], xrows[b],
                              gsems[b]).wait()
        pltpu.make_async_copy(wout_hbm.at[pl.ds(0, C * K)], nrows[b],
                              gsems[b]).wait()

    def wait_out(b):
        pltpu.make_async_copy(pos_v[b], pos_hbm.at[pl.ds(0, C)],
                              osems[b]).wait()
        pltpu.make_async_copy(neg_v[b], negout_hbm.at[pl.ds(0, C), :],
                              osems[b]).wait()

    lane = lax.iota(jnp.int32, L)
    mask_last = lane == L - 1

    def store1(ref, idxs, acc):
        plsc.store_scatter(ref, [jnp.full((L,), i, jnp.int32) for i in idxs],
                           plsc.cumsum(acc), mask=mask_last)

    def compute(c, b):
        @pl.when(c >= 2)
        def _():
            wait_out(b)

        cr, xr, nr = crows[b], xrows[b], nrows[b]
        pv, nv = pos_v[b], neg_v[b]

        def elem_body(e, carry):
            cvs = [cr[e, pl.ds(j * L, L)] for j in range(D // L)]
            pacc = cvs[0] * xr[e, pl.ds(0, L)]
            for j in range(1, D // L):
                pacc = pacc + cvs[j] * xr[e, pl.ds(j * L, L)]
            store1(pv, [e], pacc)
            for k0 in range(0, K, K // 2):
                ks = range(k0, k0 + K // 2)
                naccs = {k: cvs[0] * nr[e * K + k, pl.ds(0, L)] for k in ks}
                for j in range(1, D // L):
                    for k in ks:
                        naccs[k] = (naccs[k]
                                    + cvs[j] * nr[e * K + k, pl.ds(j * L, L)])
                for k in ks:
                    store1(nv, [e, k], naccs[k])
            return carry

        lax.fori_loop(0, C, elem_body, 0)
        base = base0 + c * C
        pltpu.async_copy(pv, pos_hbm.at[pl.ds(base, C)], osems[b])
        pltpu.async_copy(nv, negout_hbm.at[pl.ds(base, C), :], osems[b])

    fire(0, 0)

    def pair_body(i, carry):
        c0 = i * 2
        wait_gathers(0)
        fire(c0 + 1, 1)
        compute(c0, 0)
        wait_gathers(1)

        @pl.when(c0 + 2 < NCHUNK)
        def _():
            fire(c0 + 2, 0)

        compute(c0 + 1, 1)
        return carry

    lax.fori_loop(0, NCHUNK // 2, pair_body, 0)
    wait_out(0)
    wait_out(1)


@jax.jit
def _run(center, context, negflat, w_in, w_out):
    mesh = plsc.VectorSubcoreMesh(core_axis_name="c", subcore_axis_name="s",
                                  num_cores=NC, num_subcores=NS)
    kern = pl.kernel(
        _body,
        out_type=(
            jax.ShapeDtypeStruct((B,), jnp.float32),
            jax.ShapeDtypeStruct((B, K), jnp.float32),
        ),
        mesh=mesh,
        compiler_params=pltpu.CompilerParams(needs_layout_passes=False),
        scratch_types=[
            pltpu.VMEM((EPW,), jnp.int32),
            pltpu.VMEM((EPW,), jnp.int32),
            pltpu.VMEM((C, K), jnp.int32),
            pltpu.VMEM((C, K), jnp.int32),
            pltpu.VMEM((C * K,), jnp.int32),
            pltpu.VMEM((C * K,), jnp.int32),
            pltpu.VMEM((C, D), jnp.float32),
            pltpu.VMEM((C, D), jnp.float32),
            pltpu.VMEM((C, D), jnp.float32),
            pltpu.VMEM((C, D), jnp.float32),
            pltpu.VMEM((C * K, D), jnp.float32),
            pltpu.VMEM((C * K, D), jnp.float32),
            pltpu.VMEM((C,), jnp.float32),
            pltpu.VMEM((C,), jnp.float32),
            pltpu.VMEM((C, K), jnp.float32),
            pltpu.VMEM((C, K), jnp.float32),
            pltpu.SemaphoreType.DMA,
            pltpu.SemaphoreType.DMA,
            pltpu.SemaphoreType.DMA,
            pltpu.SemaphoreType.DMA,
        ],
    )
    return kern(center, context, negflat, w_in, w_out)


def kernel(center, context, negatives, W_in, W_out):
    center = center.astype(jnp.int32)
    context = context.astype(jnp.int32)
    pos, neg = _run(center, context, negatives.astype(jnp.int32),
                    W_in, W_out)
    return pos, neg


# speedup vs baseline: 1.1719x; 1.1719x over previous
import jax
import jax.numpy as jnp
from jax import lax
from jax.experimental import pallas as pl
from jax.experimental.pallas import tpu as pltpu
from jax.experimental.pallas import tpu_sc as plsc

VOCAB = 100000
D = 128
B = 16384
K = 20
NC = 2
NS = 16
NW = NC * NS
EPW = B // NW
C = 16
NCHUNK = EPW // C
NSUB = 4
SUBN = C * K // NSUB
L = 16


def _body(center_hbm, context_hbm, neg_hbm, win_hbm, wout_hbm,
          pos_hbm, negout_hbm,
          cidx, xidx, nidx2d0, nidx2d1, nflat0, nflat1,
          crows0, crows1, xrows0, xrows1,
          nrows0, nrows1, pos_v0, pos_v1, neg_v0, neg_v1,
          gsem0, gsem1, osem0, osem1, isem0, isem1):
    wid = lax.axis_index("s") * NC + lax.axis_index("c")
    base0 = wid * EPW
    nidx2d = (nidx2d0, nidx2d1)
    nflat = (nflat0, nflat1)
    crows = (crows0, crows1)
    xrows = (xrows0, xrows1)
    nrows = (nrows0, nrows1)
    pos_v = (pos_v0, pos_v1)
    neg_v = (neg_v0, neg_v1)
    gsems = (gsem0, gsem1)
    isems = (isem0, isem1)
    osems = (osem0, osem1)

    pltpu.sync_copy(center_hbm.at[pl.ds(base0, EPW)], cidx)
    pltpu.sync_copy(context_hbm.at[pl.ds(base0, EPW)], xidx)

    lane0 = lax.iota(jnp.int32, L)

    def fire(c, b):
        off = c * C
        base = base0 + off
        pltpu.make_async_copy(neg_hbm.at[pl.ds(0, C), :], nidx2d[b],
                              isems[b]).wait()
        for e in range(C):
            row = jnp.full((L,), e, jnp.int32)
            g1 = plsc.load_gather(nidx2d[b], [row, lane0])
            plsc.store_scatter(nflat[b], [e * K + lane0], g1)
            tail = 16 + (lane0 & 3)
            g2 = plsc.load_gather(nidx2d[b], [row, tail])
            plsc.store_scatter(nflat[b], [e * K + tail], g2,
                               mask=lane0 < K - 16)
        pltpu.async_copy(win_hbm.at[cidx.at[pl.ds(off, C)]],
                         crows[b], gsems[b])
        pltpu.async_copy(wout_hbm.at[xidx.at[pl.ds(off, C)]],
                         xrows[b], gsems[b])
        for j in range(NSUB):
            pltpu.async_copy(
                wout_hbm.at[nflat[b].at[pl.ds(j * SUBN, SUBN)]],
                nrows[b].at[pl.ds(j * SUBN, SUBN)], gsems[b])

    def wait_gathers(b):
        pltpu.make_async_copy(win_hbm.at[pl.ds(0, C)], crows[b],
                              gsems[b]).wait()
        pltpu.make_async_copy(wout_hbm.at[pl.ds(0, C)], xrows[b],
                              gsems[b]).wait()
        pltpu.make_async_copy(wout_hbm.at[pl.ds(0, C * K)], nrows[b],
                              gsems[b]).wait()

    def wait_out(b):
        pltpu.make_async_copy(pos_v[b], pos_hbm.at[pl.ds(0, C)],
                              osems[b]).wait()
        pltpu.make_async_copy(neg_v[b], negout_hbm.at[pl.ds(0, C), :],
                              osems[b]).wait()

    lane = lax.iota(jnp.int32, L)
    mask_last = lane == L - 1

    def store1(ref, idxs, acc):
        plsc.store_scatter(ref, [jnp.full((L,), i, jnp.int32) for i in idxs],
                           plsc.cumsum(acc), mask=mask_last)

    def compute(c, b):
        @pl.when(c >= 2)
        def _():
            wait_out(b)

        cr, xr, nr = crows[b], xrows[b], nrows[b]
        pv, nv = pos_v[b], neg_v[b]

        def elem_body(e, carry):
            cvs = [cr[e, pl.ds(j * L, L)] for j in range(D // L)]
            pacc = cvs[0] * xr[e, pl.ds(0, L)]
            for j in range(1, D // L):
                pacc = pacc + cvs[j] * xr[e, pl.ds(j * L, L)]
            store1(pv, [e], pacc)
            for k0 in range(0, K, K // 2):
                ks = range(k0, k0 + K // 2)
                naccs = {k: cvs[0] * nr[e * K + k, pl.ds(0, L)] for k in ks}
                for j in range(1, D // L):
                    for k in ks:
                        naccs[k] = (naccs[k]
                                    + cvs[j] * nr[e * K + k, pl.ds(j * L, L)])
                for k in ks:
                    store1(nv, [e, k], naccs[k])
            return carry

        lax.fori_loop(0, C, elem_body, 0)
        base = base0 + c * C
        pltpu.async_copy(pv, pos_hbm.at[pl.ds(base, C)], osems[b])
        pltpu.async_copy(nv, negout_hbm.at[pl.ds(base, C), :], osems[b])

    def stage_idx(c, b):
        pltpu.async_copy(neg_hbm.at[pl.ds(base0 + c * C, C), :],
                         nidx2d[b], isems[b])

    stage_idx(0, 0)
    fire(0, 0)
    stage_idx(1, 1)

    def pair_body(i, carry):
        c0 = i * 2
        wait_gathers(0)
        fire(c0 + 1, 1)

        @pl.when(c0 + 2 < NCHUNK)
        def _():
            stage_idx(c0 + 2, 0)

        compute(c0, 0)
        wait_gathers(1)

        @pl.when(c0 + 2 < NCHUNK)
        def _():
            fire(c0 + 2, 0)

        @pl.when(c0 + 3 < NCHUNK)
        def _():
            stage_idx(c0 + 3, 1)

        compute(c0 + 1, 1)
        return carry

    lax.fori_loop(0, NCHUNK // 2, pair_body, 0)
    wait_out(0)
    wait_out(1)


@jax.jit
def _run(center, context, negflat, w_in, w_out):
    mesh = plsc.VectorSubcoreMesh(core_axis_name="c", subcore_axis_name="s",
                                  num_cores=NC, num_subcores=NS)
    kern = pl.kernel(
        _body,
        out_type=(
            jax.ShapeDtypeStruct((B,), jnp.float32),
            jax.ShapeDtypeStruct((B, K), jnp.float32),
        ),
        mesh=mesh,
        compiler_params=pltpu.CompilerParams(needs_layout_passes=False),
        scratch_types=[
            pltpu.VMEM((EPW,), jnp.int32),
            pltpu.VMEM((EPW,), jnp.int32),
            pltpu.VMEM((C, K), jnp.int32),
            pltpu.VMEM((C, K), jnp.int32),
            pltpu.VMEM((C * K,), jnp.int32),
            pltpu.VMEM((C * K,), jnp.int32),
            pltpu.VMEM((C, D), jnp.float32),
            pltpu.VMEM((C, D), jnp.float32),
            pltpu.VMEM((C, D), jnp.float32),
            pltpu.VMEM((C, D), jnp.float32),
            pltpu.VMEM((C * K, D), jnp.float32),
            pltpu.VMEM((C * K, D), jnp.float32),
            pltpu.VMEM((C,), jnp.float32),
            pltpu.VMEM((C,), jnp.float32),
            pltpu.VMEM((C, K), jnp.float32),
            pltpu.VMEM((C, K), jnp.float32),
            pltpu.SemaphoreType.DMA,
            pltpu.SemaphoreType.DMA,
            pltpu.SemaphoreType.DMA,
            pltpu.SemaphoreType.DMA,
            pltpu.SemaphoreType.DMA,
            pltpu.SemaphoreType.DMA,
        ],
    )
    return kern(center, context, negflat, w_in, w_out)


def kernel(center, context, negatives, W_in, W_out):
    center = center.astype(jnp.int32)
    context = context.astype(jnp.int32)
    pos, neg = _run(center, context, negatives.astype(jnp.int32),
                    W_in, W_out)
    return pos, neg
